# Initial kernel scaffold; baseline (speedup 1.0000x reference)
import jax
import jax.numpy as jnp
from jax.experimental import pallas as pl


def kernel(x, edge_index, W, b):
    raise NotImplementedError("write your pallas kernel here")


# speedup vs baseline: 10.3646x; 10.3646x over previous
import functools

import jax
import jax.numpy as jnp
from jax import lax
from jax.experimental import pallas as pl
from jax.experimental.pallas import tpu as pltpu
from jax.experimental.pallas import tpu_sc as plsc

CHUNK = 128
ROWB = 256


def _deg_kernel(n_pad, cpt, nc, ns):
    mesh = plsc.VectorSubcoreMesh(core_axis_name="c", subcore_axis_name="s")
    nw = nc * ns
    ept = cpt * CHUNK

    @functools.partial(
        pl.kernel,
        mesh=mesh,
        out_type=jax.ShapeDtypeStruct((nw, n_pad), jnp.float32),
        compiler_params=pltpu.CompilerParams(needs_layout_passes=False),
        scratch_types=[
            pltpu.VMEM((ept,), jnp.int32),
            pltpu.VMEM((n_pad,), jnp.float32),
        ],
    )
    def k(dst_hbm, out_hbm, dstv, degv):
        cid = lax.axis_index("c")
        sid = lax.axis_index("s")
        wid = sid * nc + cid
        pltpu.sync_copy(dst_hbm.at[pl.ds(wid * ept, ept)], dstv)

        def z(i, c):
            degv[pl.ds(i * 16, 16)] = jnp.zeros((16,), jnp.float32)
            return c

        lax.fori_loop(0, n_pad // 16, z, 0)
        ones = jnp.ones((16,), jnp.float32)

        def step(j, c):
            idx = dstv[pl.ds(j * 16, 16)]
            plsc.addupdate_scatter(degv, [idx], ones)
            return c

        lax.fori_loop(0, ept // 16, step, 0)
        pltpu.sync_copy(degv, out_hbm.at[wid])

    return k


def _scatter_kernel(n_pad, n_chunks, cpt, nc, ns, rps, d):
    mesh = plsc.VectorSubcoreMesh(core_axis_name="c", subcore_axis_name="s")
    zsrc = n_pad - CHUNK

    @functools.partial(
        pl.kernel,
        mesh=mesh,
        out_type=jax.ShapeDtypeStruct((nc, n_pad, d), jnp.float32),
        scratch_types=[
            pltpu.VMEM((cpt, CHUNK), jnp.int32),
            pltpu.VMEM((cpt, CHUNK), jnp.int32),
            pltpu.VMEM((CHUNK, d), jnp.float32),
            pltpu.VMEM_SHARED((n_pad, d), jnp.float32),
            pltpu.SemaphoreType.DMA,
        ],
    )
    def k(g_hbm, src_hbm, dst_hbm, out_hbm, srcv, dstv, rowsv, acc, sem):
        cid = lax.axis_index("c")
        sid = lax.axis_index("s")
        wid = sid * nc + cid
        pltpu.sync_copy(src_hbm.at[pl.ds(wid * cpt, cpt)], srcv)
        pltpu.sync_copy(dst_hbm.at[pl.ds(wid * cpt, cpt)], dstv)
        for t in range(rps // CHUNK):
            pltpu.sync_copy(g_hbm.at[pl.ds(zsrc, CHUNK)],
                            acc.at[pl.ds(sid * rps + t * CHUNK, CHUNK)])
        plsc.subcore_barrier()

        def step(j, carry):
            pltpu.async_copy(g_hbm.at[srcv.at[j]], rowsv, sem).wait()
            pltpu.sync_copy(rowsv, acc.at[dstv.at[j]], add=True)
            return carry

        lax.fori_loop(0, cpt, step, 0)
        plsc.subcore_barrier()
        pltpu.sync_copy(acc.at[pl.ds(sid * rps, rps)],
                        out_hbm.at[cid, pl.ds(sid * rps, rps)])

    return k


def _linear_norm(x_pad, W, deg_parts, n_pad, d):
    def body(xb, wb, degb, gb, dinvb):
        deg = jnp.sum(degb[...], axis=0) + 1.0
        dinv = lax.rsqrt(deg)
        h = jnp.dot(xb[...], wb[...], preferred_element_type=jnp.float32)
        gb[...] = h * dinv
        dinvb[...] = dinv

    nblk = n_pad // ROWB
    return pl.pallas_call(
        body,
        grid=(nblk,),
        in_specs=[
            pl.BlockSpec((ROWB, d), lambda i: (i, 0)),
            pl.BlockSpec((d, d), lambda i: (0, 0)),
            pl.BlockSpec((deg_parts.shape[0], ROWB, 1), lambda i: (0, i, 0)),
        ],
        out_specs=[
            pl.BlockSpec((ROWB, d), lambda i: (i, 0)),
            pl.BlockSpec((ROWB, 1), lambda i: (i, 0)),
        ],
        out_shape=[
            jax.ShapeDtypeStruct((n_pad, d), jnp.float32),
            jax.ShapeDtypeStruct((n_pad, 1), jnp.float32),
        ],
    )(x_pad, W, deg_parts)


def _combine(s_parts, g, dinv, b2d, n_pad, d):
    def body(sb, gb, dinvb, bb, ob):
        s = sb[0] + sb[1]
        ob[...] = jnp.maximum((s + gb[...]) * dinvb[...] + bb[...], 0.0)

    nblk = n_pad // ROWB
    return pl.pallas_call(
        body,
        grid=(nblk,),
        in_specs=[
            pl.BlockSpec((s_parts.shape[0], ROWB, d), lambda i: (0, i, 0)),
            pl.BlockSpec((ROWB, d), lambda i: (i, 0)),
            pl.BlockSpec((ROWB, 1), lambda i: (i, 0)),
            pl.BlockSpec((1, d), lambda i: (0, 0)),
        ],
        out_specs=pl.BlockSpec((ROWB, d), lambda i: (i, 0)),
        out_shape=jax.ShapeDtypeStruct((n_pad, d), jnp.float32),
    )(s_parts, g, dinv, b2d)


def kernel(x, edge_index, W, b):
    n, d = x.shape
    e = edge_index.shape[1]

    info = plsc.get_sparse_core_info()
    nc, ns = info.num_cores, info.num_subcores
    nw = nc * ns

    rows_quantum = ns * CHUNK
    n_pad = ((n + 1 + rows_quantum - 1) // rows_quantum) * rows_quantum
    rps = n_pad // ns

    cpt = (e + nw * CHUNK - 1) // (nw * CHUNK)
    cpt = ((cpt + 7) // 8) * 8
    n_chunks = cpt * nw
    e_pad = n_chunks * CHUNK

    src = edge_index[0].astype(jnp.int32)
    dst = edge_index[1].astype(jnp.int32)
    pad = jnp.full((e_pad - e,), n, dtype=jnp.int32)
    src2d = jnp.concatenate([src, pad]).reshape(n_chunks, CHUNK)
    dst2d = jnp.concatenate([dst, pad]).reshape(n_chunks, CHUNK)

    dst1d = jnp.concatenate([dst, pad])
    deg_parts = _deg_kernel(n_pad, cpt, nc, ns)(dst1d)
    deg_parts = deg_parts.reshape(nw, n_pad, 1)

    x_pad = jnp.concatenate(
        [x, jnp.zeros((n_pad - n, d), jnp.float32)], axis=0)
    g, dinv = _linear_norm(x_pad, W, deg_parts, n_pad, d)

    s_parts = _scatter_kernel(n_pad, n_chunks, cpt, nc, ns, rps, d)(
        g, src2d, dst2d)

    out_pad = _combine(s_parts, g, dinv, b.reshape(1, d), n_pad, d)
    return out_pad[:n]


# speedup vs baseline: 11.0456x; 1.0657x over previous
import functools

import jax
import jax.numpy as jnp
from jax import lax
from jax.experimental import pallas as pl
from jax.experimental.pallas import tpu as pltpu
from jax.experimental.pallas import tpu_sc as plsc

CHUNK = 128
GROUP = 16
ROWB = 256


def _deg_kernel(n_pad, cpt, nc, ns):
    mesh = plsc.VectorSubcoreMesh(core_axis_name="c", subcore_axis_name="s")
    nw = nc * ns
    ept = cpt * CHUNK

    @functools.partial(
        pl.kernel,
        mesh=mesh,
        out_type=jax.ShapeDtypeStruct((nw, n_pad), jnp.float32),
        compiler_params=pltpu.CompilerParams(needs_layout_passes=False),
        scratch_types=[
            pltpu.VMEM((ept,), jnp.int32),
            pltpu.VMEM((n_pad,), jnp.float32),
        ],
    )
    def k(dst_hbm, out_hbm, dstv, degv):
        cid = lax.axis_index("c")
        sid = lax.axis_index("s")
        wid = sid * nc + cid
        pltpu.sync_copy(dst_hbm.at[pl.ds(wid * ept, ept)], dstv)

        def z(i, c):
            degv[pl.ds(i * 16, 16)] = jnp.zeros((16,), jnp.float32)
            return c

        lax.fori_loop(0, n_pad // 16, z, 0)
        ones = jnp.ones((16,), jnp.float32)

        def step(j, c):
            idx = dstv[pl.ds(j * 16, 16)]
            plsc.addupdate_scatter(degv, [idx], ones)
            return c

        lax.fori_loop(0, ept // 16, step, 0)
        pltpu.sync_copy(degv, out_hbm.at[wid])

    return k


def _scatter_kernel(n_pad, n_chunks, cpt, nc, ns, rps, d):
    mesh = plsc.VectorSubcoreMesh(core_axis_name="c", subcore_axis_name="s")
    zsrc = n_pad - CHUNK

    @functools.partial(
        pl.kernel,
        mesh=mesh,
        out_type=jax.ShapeDtypeStruct((nc, n_pad, d), jnp.float32),
        scratch_types=[
            pltpu.VMEM((GROUP, CHUNK), jnp.int32),
            pltpu.VMEM((cpt, CHUNK), jnp.int32),
            pltpu.VMEM((CHUNK, d), jnp.float32),
            pltpu.VMEM((CHUNK, d), jnp.float32),
            pltpu.VMEM_SHARED((n_pad, d), jnp.float32),
            pltpu.SemaphoreType.DMA,
            pltpu.SemaphoreType.DMA,
        ],
    )
    def k(g_hbm, src_hbm, dst_hbm, out_hbm, srcg, dstv, rows0, rows1, acc,
          sem0, sem1):
        cid = lax.axis_index("c")
        sid = lax.axis_index("s")
        wid = sid * nc + cid
        pltpu.sync_copy(dst_hbm.at[pl.ds(wid * cpt, cpt)], dstv)
        for t in range(rps // CHUNK):
            pltpu.sync_copy(g_hbm.at[pl.ds(zsrc, CHUNK)],
                            acc.at[pl.ds(sid * rps + t * CHUNK, CHUNK)])
        plsc.subcore_barrier()

        def group(gi, carry):
            base = gi * GROUP
            pltpu.sync_copy(src_hbm.at[pl.ds(wid * cpt + base, GROUP)], srcg)
            pltpu.async_copy(g_hbm.at[srcg.at[0]], rows0, sem0)

            def step2(i, c):
                l = i * 2
                j = base + l
                pltpu.async_copy(g_hbm.at[srcg.at[l + 1]], rows1, sem1)
                pltpu.make_async_copy(
                    g_hbm.at[srcg.at[l]], rows0, sem0).wait()
                pltpu.sync_copy(rows0, acc.at[dstv.at[j]], add=True)

                @pl.when(l + 2 < GROUP)
                def _():
                    pltpu.async_copy(g_hbm.at[srcg.at[l + 2]], rows0, sem0)

                pltpu.make_async_copy(
                    g_hbm.at[srcg.at[l + 1]], rows1, sem1).wait()
                pltpu.sync_copy(rows1, acc.at[dstv.at[j + 1]], add=True)
                return c

            lax.fori_loop(0, GROUP // 2, step2, 0)
            return carry

        lax.fori_loop(0, cpt // GROUP, group, 0)
        plsc.subcore_barrier()
        pltpu.sync_copy(acc.at[pl.ds(sid * rps, rps)],
                        out_hbm.at[cid, pl.ds(sid * rps, rps)])

    return k


def _linear_norm(x_pad, W, deg_parts, n_pad, d):
    def body(xb, wb, degb, gb, dinvb):
        deg = jnp.sum(degb[...], axis=0) + 1.0
        dinv = lax.rsqrt(deg)
        h = jnp.dot(xb[...], wb[...], preferred_element_type=jnp.float32)
        gb[...] = h * dinv
        dinvb[...] = dinv

    nblk = n_pad // ROWB
    return pl.pallas_call(
        body,
        grid=(nblk,),
        in_specs=[
            pl.BlockSpec((ROWB, d), lambda i: (i, 0)),
            pl.BlockSpec((d, d), lambda i: (0, 0)),
            pl.BlockSpec((deg_parts.shape[0], ROWB, 1), lambda i: (0, i, 0)),
        ],
        out_specs=[
            pl.BlockSpec((ROWB, d), lambda i: (i, 0)),
            pl.BlockSpec((ROWB, 1), lambda i: (i, 0)),
        ],
        out_shape=[
            jax.ShapeDtypeStruct((n_pad, d), jnp.float32),
            jax.ShapeDtypeStruct((n_pad, 1), jnp.float32),
        ],
    )(x_pad, W, deg_parts)


def _combine(s_parts, g, dinv, b2d, n_pad, d):
    def body(sb, gb, dinvb, bb, ob):
        s = sb[0] + sb[1]
        ob[...] = jnp.maximum((s + gb[...]) * dinvb[...] + bb[...], 0.0)

    nblk = n_pad // ROWB
    return pl.pallas_call(
        body,
        grid=(nblk,),
        in_specs=[
            pl.BlockSpec((s_parts.shape[0], ROWB, d), lambda i: (0, i, 0)),
            pl.BlockSpec((ROWB, d), lambda i: (i, 0)),
            pl.BlockSpec((ROWB, 1), lambda i: (i, 0)),
            pl.BlockSpec((1, d), lambda i: (0, 0)),
        ],
        out_specs=pl.BlockSpec((ROWB, d), lambda i: (i, 0)),
        out_shape=jax.ShapeDtypeStruct((n_pad, d), jnp.float32),
    )(s_parts, g, dinv, b2d)


def kernel(x, edge_index, W, b):
    n, d = x.shape
    e = edge_index.shape[1]

    info = plsc.get_sparse_core_info()
    nc, ns = info.num_cores, info.num_subcores
    nw = nc * ns

    rows_quantum = ns * CHUNK
    n_pad = ((n + 1 + rows_quantum - 1) // rows_quantum) * rows_quantum
    rps = n_pad // ns

    cpt = (e + nw * CHUNK - 1) // (nw * CHUNK)
    cpt = ((cpt + GROUP - 1) // GROUP) * GROUP
    n_chunks = cpt * nw
    e_pad = n_chunks * CHUNK

    src = edge_index[0].astype(jnp.int32)
    dst = edge_index[1].astype(jnp.int32)
    pad = jnp.full((e_pad - e,), n, dtype=jnp.int32)
    src2d = jnp.concatenate([src, pad]).reshape(n_chunks, CHUNK)
    dst2d = jnp.concatenate([dst, pad]).reshape(n_chunks, CHUNK)

    dst1d = jnp.concatenate([dst, pad])
    deg_parts = _deg_kernel(n_pad, cpt, nc, ns)(dst1d)
    deg_parts = deg_parts.reshape(nw, n_pad, 1)

    x_pad = jnp.concatenate(
        [x, jnp.zeros((n_pad - n, d), jnp.float32)], axis=0)
    g, dinv = _linear_norm(x_pad, W, deg_parts, n_pad, d)

    s_parts = _scatter_kernel(n_pad, n_chunks, cpt, nc, ns, rps, d)(
        g, src2d, dst2d)

    out_pad = _combine(s_parts, g, dinv, b.reshape(1, d), n_pad, d)
    return out_pad[:n]


# speedup vs baseline: 23.3202x; 2.1113x over previous
import functools

import jax
import jax.numpy as jnp
from jax import lax
from jax.experimental import pallas as pl
from jax.experimental.pallas import tpu as pltpu
from jax.experimental.pallas import tpu_sc as plsc

CHUNK = 128
GROUP = 16
ROWB = 256


def _deg_kernel(n_pad, cpt, nc, ns):
    mesh = plsc.VectorSubcoreMesh(core_axis_name="c", subcore_axis_name="s")
    nw = nc * ns
    ept = cpt * CHUNK

    @functools.partial(
        pl.kernel,
        mesh=mesh,
        out_type=jax.ShapeDtypeStruct((nw, n_pad), jnp.float32),
        compiler_params=pltpu.CompilerParams(needs_layout_passes=False),
        scratch_types=[
            pltpu.VMEM((ept,), jnp.int32),
            pltpu.VMEM((n_pad,), jnp.float32),
        ],
    )
    def k(dst_hbm, out_hbm, dstv, degv):
        cid = lax.axis_index("c")
        sid = lax.axis_index("s")
        wid = sid * nc + cid
        pltpu.sync_copy(dst_hbm.at[pl.ds(wid * ept, ept)], dstv)

        def z(i, c):
            degv[pl.ds(i * 16, 16)] = jnp.zeros((16,), jnp.float32)
            return c

        lax.fori_loop(0, n_pad // 16, z, 0)
        ones = jnp.ones((16,), jnp.float32)

        def step(j, c):
            idx = dstv[pl.ds(j * 16, 16)]
            plsc.addupdate_scatter(degv, [idx], ones)
            return c

        lax.fori_loop(0, ept // 16, step, 0)
        pltpu.sync_copy(degv, out_hbm.at[wid])

    return k


def _scatter_kernel(n_pad, n_chunks, cpt, nc, ns, rps, d):
    mesh = plsc.VectorSubcoreMesh(core_axis_name="c", subcore_axis_name="s")
    zsrc = n_pad - CHUNK

    @functools.partial(
        pl.kernel,
        mesh=mesh,
        out_type=jax.ShapeDtypeStruct((nc, n_pad, d), jnp.float32),
        scratch_types=[
            pltpu.VMEM((GROUP, CHUNK), jnp.int32),
            pltpu.VMEM((cpt, CHUNK), jnp.int32),
            pltpu.VMEM((CHUNK, d), jnp.float32),
            pltpu.VMEM((CHUNK, d), jnp.float32),
            pltpu.VMEM_SHARED((n_pad, d), jnp.float32),
            pltpu.SemaphoreType.DMA,
            pltpu.SemaphoreType.DMA,
        ],
    )
    def k(g_hbm, src_hbm, dst_hbm, out_hbm, srcg, dstv, rows0, rows1, acc,
          sem0, sem1):
        cid = lax.axis_index("c")
        sid = lax.axis_index("s")
        wid = sid * nc + cid
        pltpu.sync_copy(dst_hbm.at[pl.ds(wid * cpt, cpt)], dstv)
        for t in range(rps // CHUNK):
            pltpu.sync_copy(g_hbm.at[pl.ds(zsrc, CHUNK)],
                            acc.at[pl.ds(sid * rps + t * CHUNK, CHUNK)])
        plsc.subcore_barrier()

        def group(gi, carry):
            base = gi * GROUP
            pltpu.sync_copy(src_hbm.at[pl.ds(wid * cpt + base, GROUP)], srcg)
            pltpu.async_copy(g_hbm.at[srcg.at[0]], rows0, sem0)

            def step2(i, c):
                l = i * 2
                j = base + l
                pltpu.async_copy(g_hbm.at[srcg.at[l + 1]], rows1, sem1)
                pltpu.make_async_copy(
                    g_hbm.at[srcg.at[l]], rows0, sem0).wait()
                pltpu.sync_copy(rows0, acc.at[dstv.at[j]], add=True)

                @pl.when(l + 2 < GROUP)
                def _():
                    pltpu.async_copy(g_hbm.at[srcg.at[l + 2]], rows0, sem0)

                pltpu.make_async_copy(
                    g_hbm.at[srcg.at[l + 1]], rows1, sem1).wait()
                pltpu.sync_copy(rows1, acc.at[dstv.at[j + 1]], add=True)
                return c

            lax.fori_loop(0, GROUP // 2, step2, 0)
            return carry

        lax.fori_loop(0, cpt // GROUP, group, 0)
        plsc.subcore_barrier()
        pltpu.sync_copy(acc.at[pl.ds(sid * rps, rps)],
                        out_hbm.at[cid, pl.ds(sid * rps, rps)])

    return k


def _linear_norm(x_pad, W, deg_parts, n_pad, d):
    def body(xb, wb, degb, gb, dinvb):
        deg = jnp.sum(degb[...], axis=0) + 1.0
        dinv = lax.rsqrt(deg)
        h = jnp.dot(xb[...], wb[...], preferred_element_type=jnp.float32)
        gb[...] = h * dinv
        dinvb[...] = dinv

    nblk = n_pad // ROWB
    return pl.pallas_call(
        body,
        grid=(nblk,),
        in_specs=[
            pl.BlockSpec((ROWB, d), lambda i: (i, 0)),
            pl.BlockSpec((d, d), lambda i: (0, 0)),
            pl.BlockSpec((deg_parts.shape[0], ROWB, 1), lambda i: (0, i, 0)),
        ],
        out_specs=[
            pl.BlockSpec((ROWB, d), lambda i: (i, 0)),
            pl.BlockSpec((ROWB, 1), lambda i: (i, 0)),
        ],
        out_shape=[
            jax.ShapeDtypeStruct((n_pad, d), jnp.float32),
            jax.ShapeDtypeStruct((n_pad, 1), jnp.float32),
        ],
    )(x_pad, W, deg_parts)


def _combine(s_parts, g, dinv, b2d, n_pad, d):
    def body(sb, gb, dinvb, bb, ob):
        s = sb[0] + sb[1]
        ob[...] = jnp.maximum((s + gb[...]) * dinvb[...] + bb[...], 0.0)

    nblk = n_pad // ROWB
    return pl.pallas_call(
        body,
        grid=(nblk,),
        in_specs=[
            pl.BlockSpec((s_parts.shape[0], ROWB, d), lambda i: (0, i, 0)),
            pl.BlockSpec((ROWB, d), lambda i: (i, 0)),
            pl.BlockSpec((ROWB, 1), lambda i: (i, 0)),
            pl.BlockSpec((1, d), lambda i: (0, 0)),
        ],
        out_specs=pl.BlockSpec((ROWB, d), lambda i: (i, 0)),
        out_shape=jax.ShapeDtypeStruct((n_pad, d), jnp.float32),
    )(s_parts, g, dinv, b2d)


def kernel(x, edge_index, W, b):
    n, d = x.shape
    e = edge_index.shape[1]

    info = plsc.get_sparse_core_info()
    nc, ns = info.num_cores, info.num_subcores
    nw = nc * ns

    rows_quantum = ns * CHUNK
    n_pad = ((n + 1 + rows_quantum - 1) // rows_quantum) * rows_quantum
    rps = n_pad // ns

    cpt = (e + nw * CHUNK - 1) // (nw * CHUNK)
    cpt = ((cpt + GROUP - 1) // GROUP) * GROUP
    n_chunks = cpt * nw
    e_pad = n_chunks * CHUNK

    src = edge_index[0].astype(jnp.int32)
    dst = edge_index[1].astype(jnp.int32)
    pad = n + (jnp.arange(e_pad - e, dtype=jnp.int32) % (n_pad - n))
    src2d = jnp.concatenate([src, pad]).reshape(n_chunks, CHUNK)
    dst2d = jnp.concatenate([dst, pad]).reshape(n_chunks, CHUNK)

    dst1d = jnp.concatenate([dst, pad])
    deg_parts = _deg_kernel(n_pad, cpt, nc, ns)(dst1d)
    deg_parts = deg_parts.reshape(nw, n_pad, 1)

    x_pad = jnp.concatenate(
        [x, jnp.zeros((n_pad - n, d), jnp.float32)], axis=0)
    g, dinv = _linear_norm(x_pad, W, deg_parts, n_pad, d)

    s_parts = _scatter_kernel(n_pad, n_chunks, cpt, nc, ns, rps, d)(
        g, src2d, dst2d)

    out_pad = _combine(s_parts, g, dinv, b.reshape(1, d), n_pad, d)
    return out_pad[:n]


# speedup vs baseline: 23.7041x; 1.0165x over previous
import functools

import jax
import jax.numpy as jnp
from jax import lax
from jax.experimental import pallas as pl
from jax.experimental.pallas import tpu as pltpu
from jax.experimental.pallas import tpu_sc as plsc

CHUNK = 128
GROUP = 16
ROWB = 256


def _deg_kernel(n_pad, cpt, nc, ns):
    mesh = plsc.VectorSubcoreMesh(core_axis_name="c", subcore_axis_name="s")
    nw = nc * ns
    ept = cpt * CHUNK

    @functools.partial(
        pl.kernel,
        mesh=mesh,
        out_type=jax.ShapeDtypeStruct((nw, n_pad), jnp.float32),
        compiler_params=pltpu.CompilerParams(needs_layout_passes=False),
        scratch_types=[
            pltpu.VMEM((ept,), jnp.int32),
            pltpu.VMEM((n_pad,), jnp.float32),
        ],
    )
    def k(dst_hbm, out_hbm, dstv, degv):
        cid = lax.axis_index("c")
        sid = lax.axis_index("s")
        wid = sid * nc + cid
        pltpu.sync_copy(dst_hbm.at[pl.ds(wid * ept, ept)], dstv)

        def z(i, c):
            degv[pl.ds(i * 16, 16)] = jnp.zeros((16,), jnp.float32)
            return c

        lax.fori_loop(0, n_pad // 16, z, 0)
        ones = jnp.ones((16,), jnp.float32)

        def step(j, c):
            idx = dstv[pl.ds(j * 16, 16)]
            plsc.addupdate_scatter(degv, [idx], ones)
            return c

        lax.fori_loop(0, ept // 16, step, 0)
        pltpu.sync_copy(degv, out_hbm.at[wid])

    return k


def _scatter_kernel(n_pad, n_chunks, cpt, nc, ns, rps, d):
    mesh = plsc.VectorSubcoreMesh(core_axis_name="c", subcore_axis_name="s")
    zsrc = n_pad - CHUNK

    @functools.partial(
        pl.kernel,
        mesh=mesh,
        out_type=jax.ShapeDtypeStruct((nc, n_pad, d), jnp.float32),
        scratch_types=[
            pltpu.VMEM((GROUP, CHUNK), jnp.int32),
            pltpu.VMEM((cpt, CHUNK), jnp.int32),
            pltpu.VMEM((CHUNK, d), jnp.float32),
            pltpu.VMEM((CHUNK, d), jnp.float32),
            pltpu.VMEM_SHARED((n_pad, d), jnp.float32),
            pltpu.SemaphoreType.DMA,
            pltpu.SemaphoreType.DMA,
        ],
    )
    def k(g_hbm, src_hbm, dst_hbm, out_hbm, srcg, dstv, rows0, rows1, acc,
          sem0, sem1):
        cid = lax.axis_index("c")
        sid = lax.axis_index("s")
        wid = sid * nc + cid
        pltpu.sync_copy(dst_hbm.at[pl.ds(wid * cpt, cpt)], dstv)
        for t in range(rps // CHUNK):
            pltpu.sync_copy(g_hbm.at[pl.ds(zsrc, CHUNK)],
                            acc.at[pl.ds(sid * rps + t * CHUNK, CHUNK)])
        plsc.subcore_barrier()

        def group(gi, carry):
            base = gi * GROUP
            pltpu.sync_copy(src_hbm.at[pl.ds(wid * cpt + base, GROUP)], srcg)
            pltpu.async_copy(g_hbm.at[srcg.at[0]], rows0, sem0)

            def step2(i, c):
                l = i * 2
                j = base + l
                pltpu.async_copy(g_hbm.at[srcg.at[l + 1]], rows1, sem1)
                pltpu.make_async_copy(
                    g_hbm.at[srcg.at[l]], rows0, sem0).wait()
                pltpu.sync_copy(rows0, acc.at[dstv.at[j]], add=True)

                @pl.when(l + 2 < GROUP)
                def _():
                    pltpu.async_copy(g_hbm.at[srcg.at[l + 2]], rows0, sem0)

                pltpu.make_async_copy(
                    g_hbm.at[srcg.at[l + 1]], rows1, sem1).wait()
                pltpu.sync_copy(rows1, acc.at[dstv.at[j + 1]], add=True)
                return c

            lax.fori_loop(0, GROUP // 2, step2, 0)
            return carry

        lax.fori_loop(0, cpt // GROUP, group, 0)
        plsc.subcore_barrier()
        pltpu.sync_copy(acc.at[pl.ds(sid * rps, rps)],
                        out_hbm.at[cid, pl.ds(sid * rps, rps)])

    return k


def _linear_norm(x, W, deg_parts, n, n_pad, d):
    def body(xb, wb, degb, gb, dinvb):
        deg = jnp.sum(degb[...], axis=0) + 1.0
        dinv = lax.rsqrt(deg)
        rows = (pl.program_id(0) * ROWB
                + lax.broadcasted_iota(jnp.int32, (ROWB, 1), 0))
        h = jnp.dot(xb[...], wb[...], preferred_element_type=jnp.float32)
        gb[...] = jnp.where(rows < n, h * dinv, 0.0)
        dinvb[...] = dinv

    nblk = n_pad // ROWB
    return pl.pallas_call(
        body,
        grid=(nblk,),
        in_specs=[
            pl.BlockSpec((ROWB, d), lambda i: (i, 0)),
            pl.BlockSpec((d, d), lambda i: (0, 0)),
            pl.BlockSpec((deg_parts.shape[0], ROWB, 1), lambda i: (0, i, 0)),
        ],
        out_specs=[
            pl.BlockSpec((ROWB, d), lambda i: (i, 0)),
            pl.BlockSpec((ROWB, 1), lambda i: (i, 0)),
        ],
        out_shape=[
            jax.ShapeDtypeStruct((n_pad, d), jnp.float32),
            jax.ShapeDtypeStruct((n_pad, 1), jnp.float32),
        ],
    )(x, W, deg_parts)


def _combine(s_parts, g, dinv, b2d, n, n_pad, d):
    def body(sb, gb, dinvb, bb, ob):
        s = sb[0] + sb[1]
        ob[...] = jnp.maximum((s + gb[...]) * dinvb[...] + bb[...], 0.0)

    nblk = n_pad // ROWB
    return pl.pallas_call(
        body,
        grid=(nblk,),
        in_specs=[
            pl.BlockSpec((s_parts.shape[0], ROWB, d), lambda i: (0, i, 0)),
            pl.BlockSpec((ROWB, d), lambda i: (i, 0)),
            pl.BlockSpec((ROWB, 1), lambda i: (i, 0)),
            pl.BlockSpec((1, d), lambda i: (0, 0)),
        ],
        out_specs=pl.BlockSpec((ROWB, d), lambda i: (i, 0)),
        out_shape=jax.ShapeDtypeStruct((n, d), jnp.float32),
    )(s_parts, g, dinv, b2d)


def kernel(x, edge_index, W, b):
    n, d = x.shape
    e = edge_index.shape[1]

    info = plsc.get_sparse_core_info()
    nc, ns = info.num_cores, info.num_subcores
    nw = nc * ns

    rows_quantum = ns * CHUNK
    n_pad = ((n + 1 + rows_quantum - 1) // rows_quantum) * rows_quantum
    rps = n_pad // ns

    cpt = (e + nw * CHUNK - 1) // (nw * CHUNK)
    cpt = ((cpt + GROUP - 1) // GROUP) * GROUP
    n_chunks = cpt * nw
    e_pad = n_chunks * CHUNK

    src = edge_index[0].astype(jnp.int32)
    dst = edge_index[1].astype(jnp.int32)
    pad = n + (jnp.arange(e_pad - e, dtype=jnp.int32) % (n_pad - n))
    src2d = jnp.concatenate([src, pad]).reshape(n_chunks, CHUNK)
    dst2d = jnp.concatenate([dst, pad]).reshape(n_chunks, CHUNK)

    dst1d = jnp.concatenate([dst, pad])
    deg_parts = _deg_kernel(n_pad, cpt, nc, ns)(dst1d)
    deg_parts = deg_parts.reshape(nw, n_pad, 1)

    g, dinv = _linear_norm(x, W, deg_parts, n, n_pad, d)

    s_parts = _scatter_kernel(n_pad, n_chunks, cpt, nc, ns, rps, d)(
        g, src2d, dst2d)

    return _combine(s_parts, g, dinv, b.reshape(1, d), n, n_pad, d)


# speedup vs baseline: 35.7097x; 1.5065x over previous
import functools

import jax
import jax.numpy as jnp
from jax import lax
from jax.experimental import pallas as pl
from jax.experimental.pallas import tpu as pltpu
from jax.experimental.pallas import tpu_sc as plsc

CHUNK = 128
GROUP = 16
ROWB = 256


def _deg_kernel(n_pad, cpt, nc, ns):
    mesh = plsc.VectorSubcoreMesh(core_axis_name="c", subcore_axis_name="s")
    nw = nc * ns
    ept = cpt * CHUNK

    @functools.partial(
        pl.kernel,
        mesh=mesh,
        out_type=jax.ShapeDtypeStruct((nw, n_pad), jnp.float32),
        compiler_params=pltpu.CompilerParams(needs_layout_passes=False),
        scratch_types=[
            pltpu.VMEM((ept,), jnp.int32),
            pltpu.VMEM((n_pad,), jnp.float32),
        ],
    )
    def k(dst_hbm, out_hbm, dstv, degv):
        cid = lax.axis_index("c")
        sid = lax.axis_index("s")
        wid = sid * nc + cid
        pltpu.sync_copy(dst_hbm.at[pl.ds(wid * ept, ept)], dstv)

        def z(i, c):
            degv[pl.ds(i * 16, 16)] = jnp.zeros((16,), jnp.float32)
            return c

        lax.fori_loop(0, n_pad // 16, z, 0)
        ones = jnp.ones((16,), jnp.float32)

        def step(j, c):
            idx = dstv[pl.ds(j * 16, 16)]
            plsc.addupdate_scatter(degv, [idx], ones)
            return c

        lax.fori_loop(0, ept // 16, step, 0)
        pltpu.sync_copy(degv, out_hbm.at[wid])

    return k


def _scatter_kernel(n_pad, n_chunks, cpt, nc, ns, rps, d):
    mesh = plsc.VectorSubcoreMesh(core_axis_name="c", subcore_axis_name="s")
    zsrc = n_pad - CHUNK

    @functools.partial(
        pl.kernel,
        mesh=mesh,
        out_type=jax.ShapeDtypeStruct((nc, n_pad, d), jnp.float32),
        scratch_types=[
            pltpu.VMEM((GROUP, CHUNK), jnp.int32),
            pltpu.VMEM((cpt, CHUNK), jnp.int32),
            pltpu.VMEM((CHUNK, d), jnp.float32),
            pltpu.VMEM((CHUNK, d), jnp.float32),
            pltpu.VMEM_SHARED((n_pad, d), jnp.float32),
            pltpu.SemaphoreType.DMA,
            pltpu.SemaphoreType.DMA,
        ],
    )
    def k(g_hbm, src_hbm, dst_hbm, out_hbm, srcg, dstv, rows0, rows1, acc,
          sem0, sem1):
        cid = lax.axis_index("c")
        sid = lax.axis_index("s")
        wid = sid * nc + cid
        pltpu.sync_copy(dst_hbm.at[pl.ds(wid * cpt, cpt)], dstv)
        for t in range(rps // CHUNK):
            pltpu.sync_copy(g_hbm.at[pl.ds(zsrc, CHUNK)],
                            acc.at[pl.ds(sid * rps + t * CHUNK, CHUNK)])
        plsc.subcore_barrier()

        def group(gi, carry):
            base = gi * GROUP
            pltpu.sync_copy(src_hbm.at[pl.ds(wid * cpt + base, GROUP)], srcg)
            pltpu.async_copy(g_hbm.at[srcg.at[0]], rows0, sem0)

            def step2(i, c):
                l = i * 2
                j = base + l
                pltpu.async_copy(g_hbm.at[srcg.at[l + 1]], rows1, sem1)
                pltpu.make_async_copy(
                    g_hbm.at[srcg.at[l]], rows0, sem0).wait()
                pltpu.sync_copy(rows0, acc.at[dstv.at[j]], add=True)

                @pl.when(l + 2 < GROUP)
                def _():
                    pltpu.async_copy(g_hbm.at[srcg.at[l + 2]], rows0, sem0)

                pltpu.make_async_copy(
                    g_hbm.at[srcg.at[l + 1]], rows1, sem1).wait()
                pltpu.sync_copy(rows1, acc.at[dstv.at[j + 1]], add=True)
                return c

            lax.fori_loop(0, GROUP // 2, step2, 0)
            return carry

        lax.fori_loop(0, cpt // GROUP, group, 0)
        plsc.subcore_barrier()
        pltpu.sync_copy(acc.at[pl.ds(sid * rps, rps)],
                        out_hbm.at[cid, pl.ds(sid * rps, rps)])

    return k


def _linear_norm(x, W, deg_parts, n, n_pad, d):
    def body(xb, wb, degb, gb, dinvb):
        deg = jnp.sum(degb[...], axis=0) + 1.0
        dinv = lax.rsqrt(deg).reshape(ROWB, 1)
        rows = (pl.program_id(0) * ROWB
                + lax.broadcasted_iota(jnp.int32, (ROWB, 1), 0))
        h = jnp.dot(xb[...], wb[...], preferred_element_type=jnp.float32)
        gb[...] = jnp.where(rows < n, h * dinv, 0.0)
        dinvb[...] = dinv

    nblk = n_pad // ROWB
    return pl.pallas_call(
        body,
        grid=(nblk,),
        in_specs=[
            pl.BlockSpec((ROWB, d), lambda i: (i, 0)),
            pl.BlockSpec((d, d), lambda i: (0, 0)),
            pl.BlockSpec((deg_parts.shape[0], ROWB), lambda i: (0, i)),
        ],
        out_specs=[
            pl.BlockSpec((ROWB, d), lambda i: (i, 0)),
            pl.BlockSpec((ROWB, 1), lambda i: (i, 0)),
        ],
        out_shape=[
            jax.ShapeDtypeStruct((n_pad, d), jnp.float32),
            jax.ShapeDtypeStruct((n_pad, 1), jnp.float32),
        ],
    )(x, W, deg_parts)


def _combine(s_parts, g, dinv, b2d, n, n_pad, d):
    def body(sb, gb, dinvb, bb, ob):
        s = sb[0] + sb[1]
        ob[...] = jnp.maximum((s + gb[...]) * dinvb[...] + bb[...], 0.0)

    nblk = n_pad // ROWB
    return pl.pallas_call(
        body,
        grid=(nblk,),
        in_specs=[
            pl.BlockSpec((s_parts.shape[0], ROWB, d), lambda i: (0, i, 0)),
            pl.BlockSpec((ROWB, d), lambda i: (i, 0)),
            pl.BlockSpec((ROWB, 1), lambda i: (i, 0)),
            pl.BlockSpec((1, d), lambda i: (0, 0)),
        ],
        out_specs=pl.BlockSpec((ROWB, d), lambda i: (i, 0)),
        out_shape=jax.ShapeDtypeStruct((n, d), jnp.float32),
    )(s_parts, g, dinv, b2d)


def kernel(x, edge_index, W, b):
    n, d = x.shape
    e = edge_index.shape[1]

    info = plsc.get_sparse_core_info()
    nc, ns = info.num_cores, info.num_subcores
    nw = nc * ns

    rows_quantum = ns * CHUNK
    n_pad = ((n + 1 + rows_quantum - 1) // rows_quantum) * rows_quantum
    rps = n_pad // ns

    cpt = (e + nw * CHUNK - 1) // (nw * CHUNK)
    cpt = ((cpt + GROUP - 1) // GROUP) * GROUP
    n_chunks = cpt * nw
    e_pad = n_chunks * CHUNK

    src = edge_index[0].astype(jnp.int32)
    dst = edge_index[1].astype(jnp.int32)
    pad = n + (jnp.arange(e_pad - e, dtype=jnp.int32) % (n_pad - n))
    src2d = jnp.concatenate([src, pad]).reshape(n_chunks, CHUNK)
    dst2d = jnp.concatenate([dst, pad]).reshape(n_chunks, CHUNK)

    dst1d = jnp.concatenate([dst, pad])
    deg_parts = _deg_kernel(n_pad, cpt, nc, ns)(dst1d)

    g, dinv = _linear_norm(x, W, deg_parts, n, n_pad, d)

    s_parts = _scatter_kernel(n_pad, n_chunks, cpt, nc, ns, rps, d)(
        g, src2d, dst2d)

    return _combine(s_parts, g, dinv, b.reshape(1, d), n, n_pad, d)


# speedup vs baseline: 36.0105x; 1.0084x over previous
import functools

import jax
import jax.numpy as jnp
from jax import lax
from jax.experimental import pallas as pl
from jax.experimental.pallas import tpu as pltpu
from jax.experimental.pallas import tpu_sc as plsc

CHUNK = 128
GROUP = 16
ROWB = 256


def _deg_kernel(n_pad, cpt, nc, ns):
    mesh = plsc.VectorSubcoreMesh(core_axis_name="c", subcore_axis_name="s")
    nw = nc * ns
    ept = cpt * CHUNK

    @functools.partial(
        pl.kernel,
        mesh=mesh,
        out_type=jax.ShapeDtypeStruct((nw, n_pad), jnp.float32),
        compiler_params=pltpu.CompilerParams(needs_layout_passes=False),
        scratch_types=[
            pltpu.VMEM((ept,), jnp.int32),
            pltpu.VMEM((n_pad,), jnp.float32),
        ],
    )
    def k(dst_hbm, out_hbm, dstv, degv):
        cid = lax.axis_index("c")
        sid = lax.axis_index("s")
        wid = sid * nc + cid
        pltpu.sync_copy(dst_hbm.at[pl.ds(wid * ept, ept)], dstv)

        def z(i, c):
            degv[pl.ds(i * 16, 16)] = jnp.zeros((16,), jnp.float32)
            return c

        lax.fori_loop(0, n_pad // 16, z, 0)
        ones = jnp.ones((16,), jnp.float32)

        def step(j, c):
            idx = dstv[pl.ds(j * 16, 16)]
            plsc.addupdate_scatter(degv, [idx], ones)
            return c

        lax.fori_loop(0, ept // 16, step, 0)
        pltpu.sync_copy(degv, out_hbm.at[wid])

    return k


def _scatter_kernel(n_pad, n_chunks, cpt, nc, ns, rps, d):
    mesh = plsc.VectorSubcoreMesh(core_axis_name="c", subcore_axis_name="s")
    zsrc = n_pad - CHUNK

    @functools.partial(
        pl.kernel,
        mesh=mesh,
        out_type=jax.ShapeDtypeStruct((nc, n_pad, d), jnp.float32),
        scratch_types=[
            pltpu.VMEM((GROUP, CHUNK), jnp.int32),
            pltpu.VMEM((cpt, CHUNK), jnp.int32),
            pltpu.VMEM((CHUNK, d), jnp.float32),
            pltpu.VMEM((CHUNK, d), jnp.float32),
            pltpu.VMEM_SHARED((n_pad, d), jnp.float32),
            pltpu.SemaphoreType.DMA,
            pltpu.SemaphoreType.DMA,
            pltpu.SemaphoreType.DMA,
            pltpu.SemaphoreType.DMA,
        ],
    )
    def k(g_hbm, src_hbm, dst_hbm, out_hbm, srcg, dstv, rows0, rows1, acc,
          gsem0, gsem1, ssem0, ssem1):
        cid = lax.axis_index("c")
        sid = lax.axis_index("s")
        wid = sid * nc + cid
        pltpu.sync_copy(dst_hbm.at[pl.ds(wid * cpt, cpt)], dstv)
        for t in range(rps // CHUNK):
            pltpu.sync_copy(g_hbm.at[pl.ds(zsrc, CHUNK)],
                            acc.at[pl.ds(sid * rps + t * CHUNK, CHUNK)])
        plsc.subcore_barrier()

        def wait_scatter(rows, sem):
            pltpu.make_async_copy(rows, acc.at[dstv.at[0]], sem).wait()

        def wait_gather(idx, rows, sem):
            pltpu.make_async_copy(g_hbm.at[idx], rows, sem).wait()

        def group(gi, carry):
            base = gi * GROUP
            pltpu.sync_copy(src_hbm.at[pl.ds(wid * cpt + base, GROUP)], srcg)

            @pl.when(gi > 0)
            def _():
                wait_scatter(rows0, ssem0)

            pltpu.async_copy(g_hbm.at[srcg.at[0]], rows0, gsem0)

            def step2(i, c):
                l = i * 2
                j = base + l

                @pl.when(gi + i > 0)
                def _():
                    wait_scatter(rows1, ssem1)

                pltpu.async_copy(g_hbm.at[srcg.at[l + 1]], rows1, gsem1)
                wait_gather(srcg.at[l], rows0, gsem0)
                pltpu.async_copy(rows0, acc.at[dstv.at[j]], ssem0, add=True)

                @pl.when(l + 2 < GROUP)
                def _():
                    wait_scatter(rows0, ssem0)
                    pltpu.async_copy(g_hbm.at[srcg.at[l + 2]], rows0, gsem0)

                wait_gather(srcg.at[l + 1], rows1, gsem1)
                pltpu.async_copy(rows1, acc.at[dstv.at[j + 1]], ssem1,
                                 add=True)
                return c

            lax.fori_loop(0, GROUP // 2, step2, 0)
            return carry

        lax.fori_loop(0, cpt // GROUP, group, 0)
        wait_scatter(rows0, ssem0)
        wait_scatter(rows1, ssem1)
        plsc.subcore_barrier()
        pltpu.sync_copy(acc.at[pl.ds(sid * rps, rps)],
                        out_hbm.at[cid, pl.ds(sid * rps, rps)])

    return k


def _linear_norm(x, W, deg_parts, n, n_pad, d):
    def body(xb, wb, degb, gb, dinvb):
        deg = jnp.sum(degb[...], axis=0) + 1.0
        dinv = lax.rsqrt(deg).reshape(ROWB, 1)
        rows = (pl.program_id(0) * ROWB
                + lax.broadcasted_iota(jnp.int32, (ROWB, 1), 0))
        h = jnp.dot(xb[...], wb[...], preferred_element_type=jnp.float32)
        gb[...] = jnp.where(rows < n, h * dinv, 0.0)
        dinvb[...] = dinv

    nblk = n_pad // ROWB
    return pl.pallas_call(
        body,
        grid=(nblk,),
        in_specs=[
            pl.BlockSpec((ROWB, d), lambda i: (i, 0)),
            pl.BlockSpec((d, d), lambda i: (0, 0)),
            pl.BlockSpec((deg_parts.shape[0], ROWB), lambda i: (0, i)),
        ],
        out_specs=[
            pl.BlockSpec((ROWB, d), lambda i: (i, 0)),
            pl.BlockSpec((ROWB, 1), lambda i: (i, 0)),
        ],
        out_shape=[
            jax.ShapeDtypeStruct((n_pad, d), jnp.float32),
            jax.ShapeDtypeStruct((n_pad, 1), jnp.float32),
        ],
    )(x, W, deg_parts)


def _combine(s_parts, g, dinv, b2d, n, n_pad, d):
    def body(sb, gb, dinvb, bb, ob):
        s = sb[0] + sb[1]
        ob[...] = jnp.maximum((s + gb[...]) * dinvb[...] + bb[...], 0.0)

    nblk = n_pad // ROWB
    return pl.pallas_call(
        body,
        grid=(nblk,),
        in_specs=[
            pl.BlockSpec((s_parts.shape[0], ROWB, d), lambda i: (0, i, 0)),
            pl.BlockSpec((ROWB, d), lambda i: (i, 0)),
            pl.BlockSpec((ROWB, 1), lambda i: (i, 0)),
            pl.BlockSpec((1, d), lambda i: (0, 0)),
        ],
        out_specs=pl.BlockSpec((ROWB, d), lambda i: (i, 0)),
        out_shape=jax.ShapeDtypeStruct((n, d), jnp.float32),
    )(s_parts, g, dinv, b2d)


def kernel(x, edge_index, W, b):
    n, d = x.shape
    e = edge_index.shape[1]

    info = plsc.get_sparse_core_info()
    nc, ns = info.num_cores, info.num_subcores
    nw = nc * ns

    rows_quantum = ns * CHUNK
    n_pad = ((n + 1 + rows_quantum - 1) // rows_quantum) * rows_quantum
    rps = n_pad // ns

    cpt = (e + nw * CHUNK - 1) // (nw * CHUNK)
    cpt = ((cpt + GROUP - 1) // GROUP) * GROUP
    n_chunks = cpt * nw
    e_pad = n_chunks * CHUNK

    src = edge_index[0].astype(jnp.int32)
    dst = edge_index[1].astype(jnp.int32)
    pad = n + (jnp.arange(e_pad - e, dtype=jnp.int32) % (n_pad - n))
    src2d = jnp.concatenate([src, pad]).reshape(n_chunks, CHUNK)
    dst2d = jnp.concatenate([dst, pad]).reshape(n_chunks, CHUNK)

    dst1d = jnp.concatenate([dst, pad])
    deg_parts = _deg_kernel(n_pad, cpt, nc, ns)(dst1d)

    g, dinv = _linear_norm(x, W, deg_parts, n, n_pad, d)

    s_parts = _scatter_kernel(n_pad, n_chunks, cpt, nc, ns, rps, d)(
        g, src2d, dst2d)

    return _combine(s_parts, g, dinv, b.reshape(1, d), n, n_pad, d)


# speedup vs baseline: 39.7191x; 1.1030x over previous
import functools

import jax
import jax.numpy as jnp
from jax import lax
from jax.experimental import pallas as pl
from jax.experimental.pallas import tpu as pltpu
from jax.experimental.pallas import tpu_sc as plsc

CHUNK = 128
GROUP = 16
ROWB = 512


def _deg_kernel(n_pad, cpt, nc, ns):
    mesh = plsc.VectorSubcoreMesh(core_axis_name="c", subcore_axis_name="s")
    nw = nc * ns
    ept = cpt * CHUNK

    @functools.partial(
        pl.kernel,
        mesh=mesh,
        out_type=jax.ShapeDtypeStruct((nw, n_pad), jnp.float32),
        compiler_params=pltpu.CompilerParams(needs_layout_passes=False),
        scratch_types=[
            pltpu.VMEM((ept,), jnp.int32),
            pltpu.VMEM((n_pad,), jnp.float32),
        ],
    )
    def k(dst_hbm, out_hbm, dstv, degv):
        cid = lax.axis_index("c")
        sid = lax.axis_index("s")
        wid = sid * nc + cid
        pltpu.sync_copy(dst_hbm.at[pl.ds(wid * ept, ept)], dstv)

        def z(i, c):
            degv[pl.ds(i * 16, 16)] = jnp.zeros((16,), jnp.float32)
            return c

        lax.fori_loop(0, n_pad // 16, z, 0)
        ones = jnp.ones((16,), jnp.float32)

        def step(j, c):
            idx = dstv[pl.ds(j * 16, 16)]
            plsc.addupdate_scatter(degv, [idx], ones)
            return c

        lax.fori_loop(0, ept // 16, step, 0)
        pltpu.sync_copy(degv, out_hbm.at[wid])

    return k


def _scatter_kernel(n_pad, n_chunks, cpt, nc, ns, rps, d):
    mesh = plsc.VectorSubcoreMesh(core_axis_name="c", subcore_axis_name="s")
    zsrc = n_pad - CHUNK

    @functools.partial(
        pl.kernel,
        mesh=mesh,
        out_type=jax.ShapeDtypeStruct((nc, n_pad, d), jnp.float32),
        scratch_types=[
            pltpu.VMEM((GROUP, CHUNK), jnp.int32),
            pltpu.VMEM((cpt, CHUNK), jnp.int32),
            pltpu.VMEM((CHUNK, d), jnp.float32),
            pltpu.VMEM((CHUNK, d), jnp.float32),
            pltpu.VMEM_SHARED((n_pad, d), jnp.float32),
            pltpu.SemaphoreType.DMA,
            pltpu.SemaphoreType.DMA,
            pltpu.SemaphoreType.DMA,
            pltpu.SemaphoreType.DMA,
        ],
    )
    def k(g_hbm, src_hbm, dst_hbm, out_hbm, srcg, dstv, rows0, rows1, acc,
          gsem0, gsem1, ssem0, ssem1):
        cid = lax.axis_index("c")
        sid = lax.axis_index("s")
        wid = sid * nc + cid
        pltpu.sync_copy(dst_hbm.at[pl.ds(wid * cpt, cpt)], dstv)
        for t in range(rps // CHUNK):
            pltpu.sync_copy(g_hbm.at[pl.ds(zsrc, CHUNK)],
                            acc.at[pl.ds(sid * rps + t * CHUNK, CHUNK)])
        plsc.subcore_barrier()

        def wait_scatter(rows, sem):
            pltpu.make_async_copy(rows, acc.at[dstv.at[0]], sem).wait()

        def wait_gather(idx, rows, sem):
            pltpu.make_async_copy(g_hbm.at[idx], rows, sem).wait()

        def group(gi, carry):
            base = gi * GROUP
            pltpu.sync_copy(src_hbm.at[pl.ds(wid * cpt + base, GROUP)], srcg)

            @pl.when(gi > 0)
            def _():
                wait_scatter(rows0, ssem0)

            pltpu.async_copy(g_hbm.at[srcg.at[0]], rows0, gsem0)

            def step2(i, c):
                l = i * 2
                j = base + l

                @pl.when(gi + i > 0)
                def _():
                    wait_scatter(rows1, ssem1)

                pltpu.async_copy(g_hbm.at[srcg.at[l + 1]], rows1, gsem1)
                wait_gather(srcg.at[l], rows0, gsem0)
                pltpu.async_copy(rows0, acc.at[dstv.at[j]], ssem0, add=True)

                @pl.when(l + 2 < GROUP)
                def _():
                    wait_scatter(rows0, ssem0)
                    pltpu.async_copy(g_hbm.at[srcg.at[l + 2]], rows0, gsem0)

                wait_gather(srcg.at[l + 1], rows1, gsem1)
                pltpu.async_copy(rows1, acc.at[dstv.at[j + 1]], ssem1,
                                 add=True)
                return c

            lax.fori_loop(0, GROUP // 2, step2, 0)
            return carry

        lax.fori_loop(0, cpt // GROUP, group, 0)
        wait_scatter(rows0, ssem0)
        wait_scatter(rows1, ssem1)
        plsc.subcore_barrier()
        pltpu.sync_copy(acc.at[pl.ds(sid * rps, rps)],
                        out_hbm.at[cid, pl.ds(sid * rps, rps)])

    return k


def _linear_norm(x, W, deg_parts, n, n_pad, d):
    def body(xb, wb, degb, gb, dinvb):
        deg = jnp.sum(degb[...], axis=0) + 1.0
        dinv = lax.rsqrt(deg).reshape(ROWB, 1)
        rows = (pl.program_id(0) * ROWB
                + lax.broadcasted_iota(jnp.int32, (ROWB, 1), 0))
        h = jnp.dot(xb[...], wb[...], preferred_element_type=jnp.float32)
        gb[...] = jnp.where(rows < n, h * dinv, 0.0)
        dinvb[...] = dinv

    nblk = n_pad // ROWB
    return pl.pallas_call(
        body,
        grid=(nblk,),
        in_specs=[
            pl.BlockSpec((ROWB, d), lambda i: (i, 0)),
            pl.BlockSpec((d, d), lambda i: (0, 0)),
            pl.BlockSpec((deg_parts.shape[0], ROWB), lambda i: (0, i)),
        ],
        out_specs=[
            pl.BlockSpec((ROWB, d), lambda i: (i, 0)),
            pl.BlockSpec((ROWB, 1), lambda i: (i, 0)),
        ],
        out_shape=[
            jax.ShapeDtypeStruct((n_pad, d), jnp.float32),
            jax.ShapeDtypeStruct((n_pad, 1), jnp.float32),
        ],
    )(x, W, deg_parts)


def _combine(s_parts, g, dinv, b2d, n, n_pad, d):
    def body(sb, gb, dinvb, bb, ob):
        s = sb[0] + sb[1]
        ob[...] = jnp.maximum((s + gb[...]) * dinvb[...] + bb[...], 0.0)

    nblk = n_pad // ROWB
    return pl.pallas_call(
        body,
        grid=(nblk,),
        in_specs=[
            pl.BlockSpec((s_parts.shape[0], ROWB, d), lambda i: (0, i, 0)),
            pl.BlockSpec((ROWB, d), lambda i: (i, 0)),
            pl.BlockSpec((ROWB, 1), lambda i: (i, 0)),
            pl.BlockSpec((1, d), lambda i: (0, 0)),
        ],
        out_specs=pl.BlockSpec((ROWB, d), lambda i: (i, 0)),
        out_shape=jax.ShapeDtypeStruct((n, d), jnp.float32),
    )(s_parts, g, dinv, b2d)


def kernel(x, edge_index, W, b):
    n, d = x.shape
    e = edge_index.shape[1]

    info = plsc.get_sparse_core_info()
    nc, ns = info.num_cores, info.num_subcores
    nw = nc * ns

    rows_quantum = ns * CHUNK
    n_pad = ((n + 1 + rows_quantum - 1) // rows_quantum) * rows_quantum
    rps = n_pad // ns

    cpt = (e + nw * CHUNK - 1) // (nw * CHUNK)
    cpt = ((cpt + GROUP - 1) // GROUP) * GROUP
    n_chunks = cpt * nw
    e_pad = n_chunks * CHUNK

    src = edge_index[0].astype(jnp.int32)
    dst = edge_index[1].astype(jnp.int32)
    n_trash = n_pad - n
    pad = jnp.broadcast_to(
        n + jnp.arange(n_trash, dtype=jnp.int32),
        ((e_pad - e + n_trash - 1) // n_trash, n_trash)).reshape(-1)[
            :e_pad - e]
    src2d = jnp.concatenate([src, pad]).reshape(n_chunks, CHUNK)
    dst2d = jnp.concatenate([dst, pad]).reshape(n_chunks, CHUNK)

    dst1d = jnp.concatenate([dst, pad])
    deg_parts = _deg_kernel(n_pad, cpt, nc, ns)(dst1d)

    g, dinv = _linear_norm(x, W, deg_parts, n, n_pad, d)

    s_parts = _scatter_kernel(n_pad, n_chunks, cpt, nc, ns, rps, d)(
        g, src2d, dst2d)

    return _combine(s_parts, g, dinv, b.reshape(1, d), n, n_pad, d)
